# Initial kernel scaffold; baseline (speedup 1.0000x reference)
import functools

import jax
import jax.numpy as jnp
from jax import lax
from jax.experimental import pallas as pl
from jax.experimental.pallas import tpu as pltpu
from jax.experimental.pallas import tpu_sc as plsc

NC = 2
NS = 16
NW = NC * NS

CHUNK = 100
NBUF = 4


def _make_gather(total_rows: int, dim: int):
  assert total_rows % NW == 0
  rows_per_w = total_rows // NW
  assert rows_per_w % (CHUNK * NBUF) == 0
  nchunk = rows_per_w // CHUNK

  mesh = plsc.VectorSubcoreMesh(core_axis_name="c", subcore_axis_name="s")

  @functools.partial(
      pl.kernel,
      out_type=jax.ShapeDtypeStruct((total_rows, dim), jnp.float32),
      mesh=mesh,
      scratch_types=[
          pltpu.VMEM((nchunk, CHUNK), jnp.int32),
          [pltpu.VMEM((CHUNK, dim), jnp.float32) for _ in range(NBUF)],
          [pltpu.SemaphoreType.DMA for _ in range(NBUF)],
      ],
  )
  def gather_kernel(table_hbm, idx_hbm, out_hbm, idx_v, bufs, sems):
    wid = lax.axis_index("s") * NC + lax.axis_index("c")
    row_base = wid * rows_per_w

    pltpu.sync_copy(idx_hbm.at[wid], idx_v)

    def start_gather(chunk, b):
      pltpu.async_copy(table_hbm.at[idx_v.at[chunk]], bufs[b], sems[b])

    def wait_gather(b):
      pltpu.make_async_copy(bufs[b], bufs[b], sems[b]).wait()

    for b in range(NBUF):
      start_gather(b, b)

    @pl.loop(0, nchunk, step=NBUF)
    def _(g):
      for b in range(NBUF):
        i = g + b
        wait_gather(b)
        pltpu.sync_copy(bufs[b], out_hbm.at[pl.ds(row_base + i * CHUNK, CHUNK)])

        @pl.when(i + NBUF < nchunk)
        def _():
          start_gather(i + NBUF, b)

  return gather_kernel


@jax.jit
def kernel(x, table):
  batch, seq = x.shape
  dim = table.shape[1]
  total = batch * seq
  idx = x.astype(jnp.int32).reshape(NW, total // NW // CHUNK, CHUNK)
  out = _make_gather(total, dim)(table, idx)
  return out.reshape(batch, seq, dim)


# speedup vs baseline: 3.3489x; 3.3489x over previous
import functools

import jax
import jax.numpy as jnp
from jax import lax
from jax.experimental import pallas as pl
from jax.experimental.pallas import tpu as pltpu
from jax.experimental.pallas import tpu_sc as plsc

NC = 2
NS = 16
NW = NC * NS

CHUNK = 128
NBUF = 5


def _make_gather(total_rows: int, dim: int):
  assert total_rows % NW == 0
  rows_per_w = total_rows // NW
  assert rows_per_w % (CHUNK * NBUF) == 0
  nchunk = rows_per_w // CHUNK

  mesh = plsc.VectorSubcoreMesh(core_axis_name="c", subcore_axis_name="s")

  @functools.partial(
      pl.kernel,
      out_type=jax.ShapeDtypeStruct((total_rows, dim), jnp.float32),
      mesh=mesh,
      scratch_types=[
          pltpu.VMEM((nchunk, CHUNK), jnp.int32),
          [pltpu.VMEM((CHUNK, dim), jnp.float32) for _ in range(NBUF)],
          [pltpu.SemaphoreType.DMA for _ in range(NBUF)],
      ],
  )
  def gather_kernel(table_hbm, idx_hbm, out_hbm, idx_v, bufs, sems):
    wid = lax.axis_index("s") * NC + lax.axis_index("c")
    row_base = wid * rows_per_w

    pltpu.sync_copy(idx_hbm.at[wid], idx_v)

    def start_gather(chunk, b):
      pltpu.async_copy(table_hbm.at[idx_v.at[chunk]], bufs[b], sems[b])

    def wait_gather(b):
      pltpu.make_async_copy(table_hbm.at[pl.ds(0, CHUNK)], bufs[b], sems[b]).wait()

    for b in range(NBUF):
      start_gather(b, b)

    @pl.loop(0, nchunk, step=NBUF)
    def _(g):
      for b in range(NBUF):
        i = g + b
        wait_gather(b)
        pltpu.sync_copy(bufs[b], out_hbm.at[pl.ds(row_base + i * CHUNK, CHUNK)])

        @pl.when(i + NBUF < nchunk)
        def _():
          start_gather(i + NBUF, b)

  return gather_kernel


@jax.jit
def kernel(x, table):
  batch, seq = x.shape
  dim = table.shape[1]
  total = batch * seq
  idx = x.astype(jnp.int32).reshape(NW, total // NW // CHUNK, CHUNK)
  out = _make_gather(total, dim)(table, idx)
  return out.reshape(batch, seq, dim)
